# baseline (speedup 1.0000x reference)
import functools

import jax
import jax.numpy as jnp
from jax import lax
from jax.experimental import pallas as pl
from jax.experimental.pallas import tpu as pltpu
from jax.experimental.pallas import tpu_sc as plsc

N = 10000
E = 160000
NFEAT = 256
NHID = 1024
NCLASS = 256

NTILES = 16
NCORES = 2
CHUNK = 80
EPT_CH = 128
NBUF = 4
EPAD = NTILES * EPT_CH * CHUNK
NPAD = 10112
ROWS_PER_TILE = NPAD // NTILES
CW = 128


def _make_segsum(nchunks):
  cp = nchunks // NCORES
  nbuf = NBUF
  mesh = plsc.VectorSubcoreMesh(core_axis_name="c", subcore_axis_name="s")

  @functools.partial(
      pl.kernel,
      mesh=mesh,
      out_type=jax.ShapeDtypeStruct((nchunks, NPAD, CW), jnp.float32),
      scratch_types=[
          pltpu.VMEM((nbuf, 2, CHUNK), jnp.int32),
          pltpu.VMEM((nbuf, CHUNK), jnp.int32),
          pltpu.VMEM((nbuf, CHUNK), jnp.int32),
          [pltpu.VMEM((CHUNK, CW), jnp.float32) for _ in range(nbuf)],
          pltpu.VMEM_SHARED((NPAD, CW), jnp.float32),
          [pltpu.SemaphoreType.DMA for _ in range(nbuf)],
          [pltpu.SemaphoreType.DMA for _ in range(nbuf)],
          [pltpu.SemaphoreType.DMA for _ in range(nbuf)],
      ],
  )
  def segsum(table, e4, zrows, out_r, ebuf, gidx_v, dbuf, gbufs, agg_sh,
             se, sg, ss):
    c = lax.axis_index("c")
    s = lax.axis_index("s")

    def start_idx(ch, b):
      pltpu.async_copy(e4.at[s, ch], ebuf.at[b], se[b])

    def wait_idx(ch, b):
      pltpu.make_async_copy(e4.at[s, ch], ebuf.at[b], se[b]).wait()

    def start_gather(b):
      pltpu.async_copy(table.at[gidx_v.at[b]], gbufs[b], sg[b])

    def wait_gather(b):
      pltpu.make_async_copy(table.at[gidx_v.at[b]], gbufs[b], sg[b]).wait()

    def start_scatter(b):
      pltpu.async_copy(gbufs[b], agg_sh.at[dbuf.at[b]], ss[b], add=True)

    def wait_scatter(b):
      pltpu.make_async_copy(gbufs[b], agg_sh.at[dbuf.at[b]], ss[b]).wait()

    def unpack_idx(b, base):
      for k in range(CHUNK // 16):
        sl = pl.ds(k * 16, 16)
        gidx_v[b, sl] = ebuf[b, 0, sl] + base
        dbuf[b, sl] = ebuf[b, 1, sl]

    for jj in range(cp):
      j = c * cp + jj
      pltpu.sync_copy(zrows, agg_sh.at[pl.ds(s * ROWS_PER_TILE,
                                             ROWS_PER_TILE)])
      base = j * N
      plsc.subcore_barrier()

      for ch in range(nbuf - 1):
        start_idx(ch, ch)
      wait_idx(0, 0)
      unpack_idx(0, base)
      start_gather(0)

      @pl.loop(0, EPT_CH, step=nbuf)
      def chunk_body(chb):
        for bb in range(nbuf):
          ch = chb + bb
          b = bb
          b1 = (bb + 1) % nbuf
          b2 = (bb + nbuf - 1) % nbuf

          @pl.when(ch + 1 < EPT_CH)
          def _():
            wait_idx(ch + 1, b1)

            @pl.when(ch >= nbuf - 1)
            def _():
              wait_scatter(b1)

            unpack_idx(b1, base)
            start_gather(b1)

          @pl.when(ch + nbuf - 1 < EPT_CH)
          def _():
            start_idx(ch + nbuf - 1, b2)

          wait_gather(b)
          start_scatter(b)

      for ch in range(EPT_CH - nbuf, EPT_CH):
        wait_scatter(ch % nbuf)
      plsc.subcore_barrier()
      pltpu.sync_copy(
          agg_sh.at[pl.ds(s * ROWS_PER_TILE, ROWS_PER_TILE)],
          out_r.at[j, pl.ds(s * ROWS_PER_TILE, ROWS_PER_TILE)])

  return segsum


_segsum2 = _make_segsum(2)
_segsum8 = _make_segsum(8)


def _tc1_body(x_ref, agg_ref, w_ref, b_ref, out_ref):
  a = jnp.concatenate([agg_ref[0], agg_ref[1]], axis=-1)
  xa = x_ref[...] + a
  acc = jnp.dot(xa, w_ref[...], preferred_element_type=jnp.float32)
  out_ref[0] = jnp.maximum(acc + b_ref[0], 0.0)


def _tc1(x, agg1, w1, b1r):
  bn = 400
  grid = (N // bn, NHID // CW)
  return pl.pallas_call(
      _tc1_body,
      grid=grid,
      in_specs=[
          pl.BlockSpec((bn, NFEAT), lambda i, j: (i, 0)),
          pl.BlockSpec((2, bn, CW), lambda i, j: (0, i, 0)),
          pl.BlockSpec((NFEAT, CW), lambda i, j: (0, j)),
          pl.BlockSpec((1, 1, CW), lambda i, j: (j, 0, 0)),
      ],
      out_specs=pl.BlockSpec((1, bn, CW), lambda i, j: (j, i, 0)),
      out_shape=jax.ShapeDtypeStruct((NHID // CW, N, CW), jnp.float32),
  )(x, agg1, w1, b1r)


def _tc2_body(h_ref, agg_ref, w_ref, b_ref, out_ref):
  k = pl.program_id(1)

  @pl.when(k == 0)
  def _():
    out_ref[...] = jnp.broadcast_to(b_ref[...], out_ref.shape)

  ha = h_ref[0] + agg_ref[0]
  out_ref[...] += jnp.dot(ha, w_ref[...], preferred_element_type=jnp.float32)


def _tc2(h_r, agg2, w2, b2r):
  bn = 400
  grid = (N // bn, NHID // CW)
  return pl.pallas_call(
      _tc2_body,
      grid=grid,
      in_specs=[
          pl.BlockSpec((1, bn, CW), lambda i, k: (k, i, 0)),
          pl.BlockSpec((1, bn, CW), lambda i, k: (k, i, 0)),
          pl.BlockSpec((CW, NCLASS), lambda i, k: (k, 0)),
          pl.BlockSpec((1, NCLASS), lambda i, k: (0, 0)),
      ],
      out_specs=pl.BlockSpec((bn, NCLASS), lambda i, k: (i, 0)),
      out_shape=jax.ShapeDtypeStruct((N, NCLASS), jnp.float32),
  )(h_r, agg2, w2, b2r)


def kernel(x, edge_index, W1, b1, W2, b2):
  src = edge_index[0].astype(jnp.int32)
  dst = edge_index[1].astype(jnp.int32)
  pad = EPAD - E
  src3 = jnp.concatenate([src, jnp.zeros((pad,), jnp.int32)]).reshape(
      NTILES, EPT_CH, CHUNK)
  dst3 = jnp.concatenate([dst, jnp.full((pad,), N, jnp.int32)]).reshape(
      NTILES, EPT_CH, CHUNK)
  e4 = jnp.stack([src3, dst3], axis=2)
  zrows = jnp.zeros((ROWS_PER_TILE, CW), jnp.float32)

  x2d = x.reshape(N, 2, CW).transpose(1, 0, 2).reshape(2 * N, CW)
  agg1 = _segsum2(x2d, e4, zrows)
  h_r = _tc1(x, agg1, W1, b1.reshape(NHID // CW, 1, CW))
  agg2 = _segsum8(h_r.reshape(8 * N, CW), e4, zrows)
  out = _tc2(h_r, agg2, W2, b2.reshape(1, NCLASS))
  return out


# speedup vs baseline: 1.6409x; 1.6409x over previous
import functools

import jax
import jax.numpy as jnp
from jax import lax
from jax.experimental import pallas as pl
from jax.experimental.pallas import tpu as pltpu
from jax.experimental.pallas import tpu_sc as plsc

N = 10000
E = 160000
NFEAT = 256
NHID = 1024
NCLASS = 256

NTILES = 16
NCORES = 2
CW = 128


def _make_segsum(nchunks, dtype, chunk, ept_ch, nbuf, npad):
  cp = nchunks // NCORES
  rpt = npad // NTILES
  mesh = plsc.VectorSubcoreMesh(core_axis_name="c", subcore_axis_name="s")

  @functools.partial(
      pl.kernel,
      mesh=mesh,
      out_type=jax.ShapeDtypeStruct((nchunks, npad, CW), dtype),
      scratch_types=[
          pltpu.VMEM((nbuf, 2, chunk), jnp.int32),
          pltpu.VMEM((nbuf, chunk), jnp.int32),
          pltpu.VMEM((nbuf, chunk), jnp.int32),
          [pltpu.VMEM((chunk, CW), dtype) for _ in range(nbuf)],
          pltpu.VMEM_SHARED((npad, CW), dtype),
          [pltpu.SemaphoreType.DMA for _ in range(nbuf)],
          [pltpu.SemaphoreType.DMA for _ in range(nbuf)],
          [pltpu.SemaphoreType.DMA for _ in range(nbuf)],
      ],
  )
  def segsum(table, e4, zrows, out_r, ebuf, gidx_v, dbuf, gbufs, agg_sh,
             se, sg, ss):
    c = lax.axis_index("c")
    s = lax.axis_index("s")

    def start_idx(ch, b):
      pltpu.async_copy(e4.at[s, ch], ebuf.at[b], se[b])

    def wait_idx(ch, b):
      pltpu.make_async_copy(e4.at[s, ch], ebuf.at[b], se[b]).wait()

    def start_gather(b):
      pltpu.async_copy(table.at[gidx_v.at[b]], gbufs[b], sg[b])

    def wait_gather(b):
      pltpu.make_async_copy(table.at[gidx_v.at[b]], gbufs[b], sg[b]).wait()

    def start_scatter(b):
      pltpu.async_copy(gbufs[b], agg_sh.at[dbuf.at[b]], ss[b], add=True)

    def wait_scatter(b):
      pltpu.make_async_copy(gbufs[b], agg_sh.at[dbuf.at[b]], ss[b]).wait()

    def unpack_idx(b, base):
      for k in range(chunk // 16):
        sl = pl.ds(k * 16, 16)
        gidx_v[b, sl] = ebuf[b, 0, sl] + base
        dbuf[b, sl] = ebuf[b, 1, sl]

    for jj in range(cp):
      j = c * cp + jj
      pltpu.sync_copy(zrows, agg_sh.at[pl.ds(s * rpt, rpt)])
      base = j * N
      plsc.subcore_barrier()

      for ch in range(nbuf - 1):
        start_idx(ch, ch)
      wait_idx(0, 0)
      unpack_idx(0, base)
      start_gather(0)

      @pl.loop(0, ept_ch, step=nbuf)
      def chunk_body(chb):
        for bb in range(nbuf):
          ch = chb + bb
          b = bb
          b1 = (bb + 1) % nbuf
          b2 = (bb + nbuf - 1) % nbuf

          @pl.when(ch + 1 < ept_ch)
          def _():
            wait_idx(ch + 1, b1)

            @pl.when(ch >= nbuf - 1)
            def _():
              wait_scatter(b1)

            unpack_idx(b1, base)
            start_gather(b1)

          @pl.when(ch + nbuf - 1 < ept_ch)
          def _():
            start_idx(ch + nbuf - 1, b2)

          wait_gather(b)
          start_scatter(b)

      for ch in range(ept_ch - nbuf, ept_ch):
        wait_scatter(ch % nbuf)
      plsc.subcore_barrier()
      pltpu.sync_copy(
          agg_sh.at[pl.ds(s * rpt, rpt)],
          out_r.at[j, pl.ds(s * rpt, rpt)])

  return segsum


CHUNK1, EPT1, NPAD1 = 112, 90, 10112
_segsum2 = _make_segsum(2, jnp.float32, CHUNK1, EPT1, 3, NPAD1)
CHUNK2, EPT2, NPAD2 = 112, 90, 10112
_segsum8 = _make_segsum(8, jnp.float32, CHUNK2, EPT2, 3, NPAD2)


def _pack_edges(src, dst, chunk, ept_ch):
  epad = NTILES * ept_ch * chunk
  src_p = jnp.concatenate([src, jnp.zeros((epad - E,), jnp.int32)])
  dst_p = jnp.concatenate([dst, jnp.full((epad - E,), N, jnp.int32)])
  return jnp.stack([src_p.reshape(NTILES, ept_ch, chunk),
                    dst_p.reshape(NTILES, ept_ch, chunk)], axis=2)


def _tc1_body(x_ref, agg_ref, w_ref, b_ref, out_ref):
  a = jnp.concatenate([agg_ref[0], agg_ref[1]], axis=-1)
  xa = (x_ref[...] + a).astype(jnp.bfloat16)
  acc = jnp.dot(xa, w_ref[...], preferred_element_type=jnp.float32)
  out_ref[0] = jnp.maximum(acc + b_ref[0], 0.0)


def _tc1(x, agg1, w1, b1r):
  bn = 400
  grid = (N // bn, NHID // CW)
  return pl.pallas_call(
      _tc1_body,
      grid=grid,
      in_specs=[
          pl.BlockSpec((bn, NFEAT), lambda i, j: (i, 0)),
          pl.BlockSpec((2, bn, CW), lambda i, j: (0, i, 0)),
          pl.BlockSpec((NFEAT, CW), lambda i, j: (0, j)),
          pl.BlockSpec((1, 1, CW), lambda i, j: (j, 0, 0)),
      ],
      out_specs=pl.BlockSpec((1, bn, CW), lambda i, j: (j, i, 0)),
      out_shape=jax.ShapeDtypeStruct((NHID // CW, N, CW), jnp.float32),
  )(x, agg1, w1, b1r)


def _tc2_body(h_ref, agg_ref, w_ref, b_ref, out_ref):
  k = pl.program_id(1)

  @pl.when(k == 0)
  def _():
    out_ref[...] = jnp.broadcast_to(b_ref[...], out_ref.shape)

  ha = (h_ref[0] + agg_ref[0]).astype(jnp.bfloat16)
  out_ref[...] += jnp.dot(ha, w_ref[...], preferred_element_type=jnp.float32)


def _tc2(h_r, agg2, w2, b2r):
  bn = 400
  grid = (N // bn, NHID // CW)
  return pl.pallas_call(
      _tc2_body,
      grid=grid,
      in_specs=[
          pl.BlockSpec((1, bn, CW), lambda i, k: (k, i, 0)),
          pl.BlockSpec((1, bn, CW), lambda i, k: (k, i, 0)),
          pl.BlockSpec((CW, NCLASS), lambda i, k: (k, 0)),
          pl.BlockSpec((1, NCLASS), lambda i, k: (0, 0)),
      ],
      out_specs=pl.BlockSpec((bn, NCLASS), lambda i, k: (i, 0)),
      out_shape=jax.ShapeDtypeStruct((N, NCLASS), jnp.float32),
  )(h_r, agg2, w2, b2r)


def kernel(x, edge_index, W1, b1, W2, b2):
  src = edge_index[0].astype(jnp.int32)
  dst = edge_index[1].astype(jnp.int32)
  e4a = _pack_edges(src, dst, CHUNK1, EPT1)
  e4b = _pack_edges(src, dst, CHUNK2, EPT2)
  zrows1 = jnp.zeros((NPAD1 // NTILES, CW), jnp.float32)

  x2d = x.reshape(N, 2, CW).transpose(1, 0, 2).reshape(2 * N, CW)
  agg1 = _segsum2(x2d, e4a, zrows1)
  h_r = _tc1(x, agg1, W1.astype(jnp.bfloat16),
             b1.reshape(NHID // CW, 1, CW))
  agg2 = _segsum8(h_r.reshape(8 * N, CW), e4b, zrows1)
  out = _tc2(h_r, agg2, W2.astype(jnp.bfloat16), b2.reshape(1, NCLASS))
  return out


# speedup vs baseline: 1.8692x; 1.1392x over previous
import functools

import jax
import jax.numpy as jnp
from jax import lax
from jax.experimental import pallas as pl
from jax.experimental.pallas import tpu as pltpu
from jax.experimental.pallas import tpu_sc as plsc

N = 10000
E = 160000
NFEAT = 256
NHID = 1024
NCLASS = 256

NTILES = 16
NCORES = 2
CW = 128


def _make_segsum(nchunks, dtype, chunk, ept_ch, nbuf, npad):
  cp = nchunks // NCORES
  rpt = npad // NTILES
  mesh = plsc.VectorSubcoreMesh(core_axis_name="c", subcore_axis_name="s")

  @functools.partial(
      pl.kernel,
      mesh=mesh,
      out_type=jax.ShapeDtypeStruct((nchunks, npad, CW), dtype),
      scratch_types=[
          pltpu.VMEM((nbuf, 2, chunk), jnp.int32),
          pltpu.VMEM((nbuf, chunk), jnp.int32),
          pltpu.VMEM((nbuf, chunk), jnp.int32),
          [pltpu.VMEM((chunk, CW), dtype) for _ in range(nbuf)],
          pltpu.VMEM_SHARED((npad, CW), dtype),
          [pltpu.SemaphoreType.DMA for _ in range(nbuf)],
          [pltpu.SemaphoreType.DMA for _ in range(nbuf)],
          [pltpu.SemaphoreType.DMA for _ in range(nbuf)],
      ],
  )
  def segsum(table, e4, zrows, out_r, ebuf, gidx_v, dbuf, gbufs, agg_sh,
             se, sg, ss):
    c = lax.axis_index("c")
    s = lax.axis_index("s")

    def start_idx(ch, b):
      pltpu.async_copy(e4.at[s, ch], ebuf.at[b], se[b])

    def wait_idx(ch, b):
      pltpu.make_async_copy(e4.at[s, ch], ebuf.at[b], se[b]).wait()

    def start_gather(b):
      pltpu.async_copy(table.at[gidx_v.at[b]], gbufs[b], sg[b])

    def wait_gather(b):
      pltpu.make_async_copy(table.at[gidx_v.at[b]], gbufs[b], sg[b]).wait()

    def start_scatter(b):
      pltpu.async_copy(gbufs[b], agg_sh.at[dbuf.at[b]], ss[b], add=True)

    def wait_scatter(b):
      pltpu.make_async_copy(gbufs[b], agg_sh.at[dbuf.at[b]], ss[b]).wait()

    def unpack_idx(b, base):
      for k in range(chunk // 16):
        sl = pl.ds(k * 16, 16)
        gidx_v[b, sl] = ebuf[b, 0, sl] + base
        dbuf[b, sl] = ebuf[b, 1, sl]

    for jj in range(cp):
      j = c * cp + jj
      pltpu.sync_copy(zrows, agg_sh.at[pl.ds(s * rpt, rpt)])
      base = j * N
      plsc.subcore_barrier()

      for ch in range(nbuf - 1):
        start_idx(ch, ch)
      wait_idx(0, 0)
      unpack_idx(0, base)
      start_gather(0)

      @pl.loop(0, ept_ch, step=nbuf)
      def chunk_body(chb):
        for bb in range(nbuf):
          ch = chb + bb
          b = bb
          b1 = (bb + 1) % nbuf
          b2 = (bb + nbuf - 1) % nbuf

          @pl.when(ch + 1 < ept_ch)
          def _():
            wait_idx(ch + 1, b1)

            @pl.when(ch >= nbuf - 1)
            def _():
              wait_scatter(b1)

            unpack_idx(b1, base)
            start_gather(b1)

          @pl.when(ch + nbuf - 1 < ept_ch)
          def _():
            start_idx(ch + nbuf - 1, b2)

          wait_gather(b)
          start_scatter(b)

      for ch in range(ept_ch - nbuf, ept_ch):
        wait_scatter(ch % nbuf)
      plsc.subcore_barrier()
      pltpu.sync_copy(
          agg_sh.at[pl.ds(s * rpt, rpt)],
          out_r.at[j, pl.ds(s * rpt, rpt)])

  return segsum


CHUNK1, EPT1, NPAD1 = 112, 90, 10112
_segsum2 = _make_segsum(2, jnp.float32, CHUNK1, EPT1, 3, NPAD1)
CHUNK2, EPT2, NPAD2 = 112, 90, 10112
_segsum8 = _make_segsum(8, jnp.float32, CHUNK2, EPT2, 3, NPAD2)


def _pack_edges(src, dst, chunk, ept_ch):
  epad = NTILES * ept_ch * chunk
  src_p = jnp.concatenate([src, jnp.zeros((epad - E,), jnp.int32)])
  dst_p = jnp.concatenate([dst, jnp.full((epad - E,), N, jnp.int32)])
  return jnp.stack([src_p.reshape(NTILES, ept_ch, chunk),
                    dst_p.reshape(NTILES, ept_ch, chunk)], axis=2)


def _tc1_body(x_ref, agg_ref, w_ref, b_ref, out_ref):
  a = jnp.concatenate([agg_ref[0], agg_ref[1]], axis=-1)
  xa = (x_ref[...] + a).astype(jnp.bfloat16)
  acc = jnp.dot(xa, w_ref[...], preferred_element_type=jnp.float32)
  out_ref[0] = jnp.maximum(acc + b_ref[0], 0.0)


def _tc1(x, agg1, w1, b1r):
  bn = 1000
  grid = (N // bn, NHID // CW)
  return pl.pallas_call(
      _tc1_body,
      grid=grid,
      in_specs=[
          pl.BlockSpec((bn, NFEAT), lambda i, j: (i, 0)),
          pl.BlockSpec((2, bn, CW), lambda i, j: (0, i, 0)),
          pl.BlockSpec((NFEAT, CW), lambda i, j: (0, j)),
          pl.BlockSpec((1, 1, CW), lambda i, j: (j, 0, 0)),
      ],
      out_specs=pl.BlockSpec((1, bn, CW), lambda i, j: (j, i, 0)),
      out_shape=jax.ShapeDtypeStruct((NHID // CW, N, CW), jnp.float32),
  )(x, agg1, w1, b1r)


def _tc2_body(h_ref, agg_ref, w_ref, b_ref, out_ref):
  k = pl.program_id(1)

  @pl.when(k == 0)
  def _():
    out_ref[...] = jnp.broadcast_to(b_ref[...], out_ref.shape)

  ha = (h_ref[0] + agg_ref[0]).astype(jnp.bfloat16)
  out_ref[...] += jnp.dot(ha, w_ref[...], preferred_element_type=jnp.float32)


def _tc2(h_r, agg2, w2, b2r):
  bn = 1000
  grid = (N // bn, NHID // CW)
  return pl.pallas_call(
      _tc2_body,
      grid=grid,
      in_specs=[
          pl.BlockSpec((1, bn, CW), lambda i, k: (k, i, 0)),
          pl.BlockSpec((1, bn, CW), lambda i, k: (k, i, 0)),
          pl.BlockSpec((CW, NCLASS), lambda i, k: (k, 0)),
          pl.BlockSpec((1, NCLASS), lambda i, k: (0, 0)),
      ],
      out_specs=pl.BlockSpec((bn, NCLASS), lambda i, k: (i, 0)),
      out_shape=jax.ShapeDtypeStruct((N, NCLASS), jnp.float32),
  )(h_r, agg2, w2, b2r)


def kernel(x, edge_index, W1, b1, W2, b2):
  src = edge_index[0].astype(jnp.int32)
  dst = edge_index[1].astype(jnp.int32)
  e4a = _pack_edges(src, dst, CHUNK1, EPT1)
  e4b = _pack_edges(src, dst, CHUNK2, EPT2)
  zrows1 = jnp.zeros((NPAD1 // NTILES, CW), jnp.float32)

  x2d = x.reshape(N, 2, CW).transpose(1, 0, 2).reshape(2 * N, CW)
  agg1 = _segsum2(x2d, e4a, zrows1)
  h_r = _tc1(x, agg1, W1.astype(jnp.bfloat16),
             b1.reshape(NHID // CW, 1, CW))
  agg2 = _segsum8(h_r.reshape(8 * N, CW), e4b, zrows1)
  out = _tc2(h_r, agg2, W2.astype(jnp.bfloat16), b2.reshape(1, NCLASS))
  return out


# speedup vs baseline: 1.9650x; 1.0512x over previous
import functools

import jax
import jax.numpy as jnp
from jax import lax
from jax.experimental import pallas as pl
from jax.experimental.pallas import tpu as pltpu
from jax.experimental.pallas import tpu_sc as plsc

N = 10000
E = 160000
NFEAT = 256
NHID = 1024
NCLASS = 256

NTILES = 16
NCORES = 2
CW = 128


def _make_segsum(nchunks, dtype, chunk, ept_ch, nbuf, npad):
  cp = nchunks // NCORES
  rpt = npad // NTILES
  mesh = plsc.VectorSubcoreMesh(core_axis_name="c", subcore_axis_name="s")

  @functools.partial(
      pl.kernel,
      mesh=mesh,
      out_type=jax.ShapeDtypeStruct((nchunks, npad, CW), dtype),
      scratch_types=[
          pltpu.VMEM((nbuf, 2, chunk), jnp.int32),
          pltpu.VMEM((nbuf, chunk), jnp.int32),
          pltpu.VMEM((nbuf, chunk), jnp.int32),
          [pltpu.VMEM((chunk, CW), dtype) for _ in range(nbuf)],
          pltpu.VMEM_SHARED((npad, CW), dtype),
          [pltpu.SemaphoreType.DMA for _ in range(nbuf)],
          [pltpu.SemaphoreType.DMA for _ in range(nbuf)],
          [pltpu.SemaphoreType.DMA for _ in range(nbuf)],
      ],
  )
  def segsum(table, e4, zrows, out_r, ebuf, gidx_v, dbuf, gbufs, agg_sh,
             se, sg, ss):
    c = lax.axis_index("c")
    s = lax.axis_index("s")

    def start_idx(ch, b):
      pltpu.async_copy(e4.at[s, ch], ebuf.at[b], se[b])

    def wait_idx(ch, b):
      pltpu.make_async_copy(e4.at[s, ch], ebuf.at[b], se[b]).wait()

    def start_gather(b):
      pltpu.async_copy(table.at[gidx_v.at[b]], gbufs[b], sg[b])

    def wait_gather(b):
      pltpu.make_async_copy(table.at[gidx_v.at[b]], gbufs[b], sg[b]).wait()

    def start_scatter(b):
      pltpu.async_copy(gbufs[b], agg_sh.at[dbuf.at[b]], ss[b], add=True)

    def wait_scatter(b):
      pltpu.make_async_copy(gbufs[b], agg_sh.at[dbuf.at[b]], ss[b]).wait()

    def unpack_idx(b, base):
      for k in range(chunk // 16):
        sl = pl.ds(k * 16, 16)
        gidx_v[b, sl] = ebuf[b, 0, sl] + base
        dbuf[b, sl] = ebuf[b, 1, sl]

    for jj in range(cp):
      j = c * cp + jj
      pltpu.sync_copy(zrows, agg_sh.at[pl.ds(s * rpt, rpt)])
      base = j * N
      plsc.subcore_barrier()

      for ch in range(nbuf - 1):
        start_idx(ch, ch)
      wait_idx(0, 0)
      unpack_idx(0, base)
      start_gather(0)

      @pl.loop(0, ept_ch, step=nbuf)
      def chunk_body(chb):
        for bb in range(nbuf):
          ch = chb + bb
          b = bb
          b1 = (bb + 1) % nbuf
          b2 = (bb + nbuf - 1) % nbuf

          @pl.when(ch + 1 < ept_ch)
          def _():
            wait_idx(ch + 1, b1)

            @pl.when(ch >= nbuf - 1)
            def _():
              wait_scatter(b1)

            unpack_idx(b1, base)
            start_gather(b1)

          @pl.when(ch + nbuf - 1 < ept_ch)
          def _():
            start_idx(ch + nbuf - 1, b2)

          wait_gather(b)
          start_scatter(b)

      for ch in range(ept_ch - nbuf, ept_ch):
        wait_scatter(ch % nbuf)
      plsc.subcore_barrier()
      pltpu.sync_copy(
          agg_sh.at[pl.ds(s * rpt, rpt)],
          out_r.at[j, pl.ds(s * rpt, rpt)])

  return segsum


CHUNK1, EPT1, NPAD1 = 112, 90, 10112
_segsum2 = _make_segsum(2, jnp.float32, CHUNK1, EPT1, 3, NPAD1)
CHUNK2, EPT2, NPAD2 = 112, 90, 10112
_segsum8 = _make_segsum(8, jnp.float32, CHUNK2, EPT2, 3, NPAD2)


def _pack_edges(src, dst, chunk, ept_ch):
  epad = NTILES * ept_ch * chunk
  src_p = jnp.concatenate([src, jnp.zeros((epad - E,), jnp.int32)])
  dst_p = jnp.concatenate([dst, jnp.full((epad - E,), N, jnp.int32)])
  return jnp.stack([src_p.reshape(NTILES, ept_ch, chunk),
                    dst_p.reshape(NTILES, ept_ch, chunk)], axis=2)


def _tc1_body(x_ref, agg_ref, w_ref, b_ref, out_ref):
  a = jnp.concatenate([agg_ref[0], agg_ref[1]], axis=-1)
  xa = (x_ref[...] + a).astype(jnp.bfloat16)
  acc = jnp.dot(xa, w_ref[...], preferred_element_type=jnp.float32)
  out_ref[0] = jnp.maximum(acc + b_ref[0], 0.0)


def _tc1(x, agg1, w1, b1r):
  bn = 2000
  grid = (N // bn, NHID // CW)
  return pl.pallas_call(
      _tc1_body,
      grid=grid,
      in_specs=[
          pl.BlockSpec((bn, NFEAT), lambda i, j: (i, 0)),
          pl.BlockSpec((2, bn, CW), lambda i, j: (0, i, 0)),
          pl.BlockSpec((NFEAT, CW), lambda i, j: (0, j)),
          pl.BlockSpec((1, 1, CW), lambda i, j: (j, 0, 0)),
      ],
      out_specs=pl.BlockSpec((1, bn, CW), lambda i, j: (j, i, 0)),
      out_shape=jax.ShapeDtypeStruct((NHID // CW, N, CW), jnp.float32),
  )(x, agg1, w1, b1r)


def _tc2_body(h_ref, agg_ref, w_ref, b_ref, out_ref):
  k = pl.program_id(1)

  @pl.when(k == 0)
  def _():
    out_ref[...] = jnp.broadcast_to(b_ref[...], out_ref.shape)

  ha = (h_ref[0] + agg_ref[0]).astype(jnp.bfloat16)
  out_ref[...] += jnp.dot(ha, w_ref[...], preferred_element_type=jnp.float32)


def _tc2(h_r, agg2, w2, b2r):
  bn = 2000
  grid = (N // bn, NHID // CW)
  return pl.pallas_call(
      _tc2_body,
      grid=grid,
      in_specs=[
          pl.BlockSpec((1, bn, CW), lambda i, k: (k, i, 0)),
          pl.BlockSpec((1, bn, CW), lambda i, k: (k, i, 0)),
          pl.BlockSpec((CW, NCLASS), lambda i, k: (k, 0)),
          pl.BlockSpec((1, NCLASS), lambda i, k: (0, 0)),
      ],
      out_specs=pl.BlockSpec((bn, NCLASS), lambda i, k: (i, 0)),
      out_shape=jax.ShapeDtypeStruct((N, NCLASS), jnp.float32),
  )(h_r, agg2, w2, b2r)


def kernel(x, edge_index, W1, b1, W2, b2):
  src = edge_index[0].astype(jnp.int32)
  dst = edge_index[1].astype(jnp.int32)
  e4a = _pack_edges(src, dst, CHUNK1, EPT1)
  e4b = _pack_edges(src, dst, CHUNK2, EPT2)
  zrows1 = jnp.zeros((NPAD1 // NTILES, CW), jnp.float32)

  x2d = x.reshape(N, 2, CW).transpose(1, 0, 2).reshape(2 * N, CW)
  agg1 = _segsum2(x2d, e4a, zrows1)
  h_r = _tc1(x, agg1, W1.astype(jnp.bfloat16),
             b1.reshape(NHID // CW, 1, CW))
  agg2 = _segsum8(h_r.reshape(8 * N, CW), e4b, zrows1)
  out = _tc2(h_r, agg2, W2.astype(jnp.bfloat16), b2.reshape(1, NCLASS))
  return out


# speedup vs baseline: 2.0559x; 1.0463x over previous
import functools

import jax
import jax.numpy as jnp
from jax import lax
from jax.experimental import pallas as pl
from jax.experimental.pallas import tpu as pltpu
from jax.experimental.pallas import tpu_sc as plsc

N = 10000
E = 160000
NFEAT = 256
NHID = 1024
NCLASS = 256

NTILES = 16
NCORES = 2
CW = 128


def _make_segsum(nchunks, dtype, chunk, ept_ch, nbuf, npad):
  cp = nchunks // NCORES
  rpt = npad // NTILES
  mesh = plsc.VectorSubcoreMesh(core_axis_name="c", subcore_axis_name="s")

  @functools.partial(
      pl.kernel,
      mesh=mesh,
      out_type=jax.ShapeDtypeStruct((nchunks, npad, CW), dtype),
      scratch_types=[
          pltpu.VMEM((nbuf, 2, chunk), jnp.int32),
          pltpu.VMEM((nbuf, chunk), jnp.int32),
          pltpu.VMEM((nbuf, chunk), jnp.int32),
          [pltpu.VMEM((chunk, CW), dtype) for _ in range(nbuf)],
          pltpu.VMEM_SHARED((npad, CW), dtype),
          [pltpu.SemaphoreType.DMA for _ in range(nbuf)],
          [pltpu.SemaphoreType.DMA for _ in range(nbuf)],
          [pltpu.SemaphoreType.DMA for _ in range(nbuf)],
      ],
  )
  def segsum(table, e4, zrows, out_r, ebuf, gidx_v, dbuf, gbufs, agg_sh,
             se, sg, ss):
    c = lax.axis_index("c")
    s = lax.axis_index("s")

    def start_idx(ch, b):
      pltpu.async_copy(e4.at[s, ch], ebuf.at[b], se[b])

    def wait_idx(ch, b):
      pltpu.make_async_copy(e4.at[s, ch], ebuf.at[b], se[b]).wait()

    def start_gather(b):
      pltpu.async_copy(table.at[gidx_v.at[b]], gbufs[b], sg[b])

    def wait_gather(b):
      pltpu.make_async_copy(table.at[gidx_v.at[b]], gbufs[b], sg[b]).wait()

    def start_scatter(b):
      pltpu.async_copy(gbufs[b], agg_sh.at[dbuf.at[b]], ss[b], add=True)

    def wait_scatter(b):
      pltpu.make_async_copy(gbufs[b], agg_sh.at[dbuf.at[b]], ss[b]).wait()

    def unpack_idx(b, base):
      for k in range(chunk // 16):
        sl = pl.ds(k * 16, 16)
        gidx_v[b, sl] = ebuf[b, 0, sl] + base
        dbuf[b, sl] = ebuf[b, 1, sl]

    for jj in range(cp):
      j = c * cp + jj
      pltpu.sync_copy(zrows, agg_sh.at[pl.ds(s * rpt, rpt)])
      base = j * N
      plsc.subcore_barrier()

      for ch in range(nbuf - 1):
        start_idx(ch, ch)
      wait_idx(0, 0)
      unpack_idx(0, base)
      start_gather(0)

      @pl.loop(0, ept_ch, step=nbuf)
      def chunk_body(chb):
        for bb in range(nbuf):
          ch = chb + bb
          b = bb
          b1 = (bb + 1) % nbuf
          b2 = (bb + nbuf - 1) % nbuf

          @pl.when(ch + 1 < ept_ch)
          def _():
            wait_idx(ch + 1, b1)

            @pl.when(ch >= nbuf - 1)
            def _():
              wait_scatter(b1)

            unpack_idx(b1, base)
            start_gather(b1)

          @pl.when(ch + nbuf - 1 < ept_ch)
          def _():
            start_idx(ch + nbuf - 1, b2)

          wait_gather(b)
          start_scatter(b)

      for ch in range(ept_ch - nbuf, ept_ch):
        wait_scatter(ch % nbuf)
      plsc.subcore_barrier()
      pltpu.sync_copy(
          agg_sh.at[pl.ds(s * rpt, rpt)],
          out_r.at[j, pl.ds(s * rpt, rpt)])

  return segsum


CHUNK1, EPT1, NPAD1 = 112, 90, 10112
_segsum2 = _make_segsum(2, jnp.float32, CHUNK1, EPT1, 3, NPAD1)
CHUNK2, EPT2, NPAD2 = 112, 90, 10112
_segsum8 = _make_segsum(8, jnp.float32, CHUNK2, EPT2, 3, NPAD2)


def _pack_edges(src, dst, chunk, ept_ch):
  epad = NTILES * ept_ch * chunk
  src_p = jnp.concatenate([src, jnp.zeros((epad - E,), jnp.int32)])
  dst_p = jnp.concatenate([dst, jnp.full((epad - E,), N, jnp.int32)])
  return jnp.stack([src_p.reshape(NTILES, ept_ch, chunk),
                    dst_p.reshape(NTILES, ept_ch, chunk)], axis=2)


def _tc1_body(x_ref, agg_ref, w_ref, b_ref, out_ref):
  a = jnp.concatenate([agg_ref[0], agg_ref[1]], axis=-1)
  xa = (x_ref[...] + a).astype(jnp.bfloat16)
  acc = jnp.dot(xa, w_ref[...], preferred_element_type=jnp.float32)
  out_ref[0] = jnp.maximum(acc + b_ref[0], 0.0)


def _tc1(x, agg1, w1, b1r):
  bn = 10000
  grid = (N // bn, NHID // CW)
  return pl.pallas_call(
      _tc1_body,
      grid=grid,
      in_specs=[
          pl.BlockSpec((bn, NFEAT), lambda i, j: (i, 0)),
          pl.BlockSpec((2, bn, CW), lambda i, j: (0, i, 0)),
          pl.BlockSpec((NFEAT, CW), lambda i, j: (0, j)),
          pl.BlockSpec((1, 1, CW), lambda i, j: (j, 0, 0)),
      ],
      out_specs=pl.BlockSpec((1, bn, CW), lambda i, j: (j, i, 0)),
      out_shape=jax.ShapeDtypeStruct((NHID // CW, N, CW), jnp.float32),
  )(x, agg1, w1, b1r)


def _tc2_body(h_ref, agg_ref, w_ref, b_ref, out_ref):
  k = pl.program_id(1)

  @pl.when(k == 0)
  def _():
    out_ref[...] = jnp.broadcast_to(b_ref[...], out_ref.shape)

  ha = (h_ref[0] + agg_ref[0]).astype(jnp.bfloat16)
  out_ref[...] += jnp.dot(ha, w_ref[...], preferred_element_type=jnp.float32)


def _tc2(h_r, agg2, w2, b2r):
  bn = 10000
  grid = (N // bn, NHID // CW)
  return pl.pallas_call(
      _tc2_body,
      grid=grid,
      in_specs=[
          pl.BlockSpec((1, bn, CW), lambda i, k: (k, i, 0)),
          pl.BlockSpec((1, bn, CW), lambda i, k: (k, i, 0)),
          pl.BlockSpec((CW, NCLASS), lambda i, k: (k, 0)),
          pl.BlockSpec((1, NCLASS), lambda i, k: (0, 0)),
      ],
      out_specs=pl.BlockSpec((bn, NCLASS), lambda i, k: (i, 0)),
      out_shape=jax.ShapeDtypeStruct((N, NCLASS), jnp.float32),
  )(h_r, agg2, w2, b2r)


def kernel(x, edge_index, W1, b1, W2, b2):
  src = edge_index[0].astype(jnp.int32)
  dst = edge_index[1].astype(jnp.int32)
  e4a = _pack_edges(src, dst, CHUNK1, EPT1)
  e4b = _pack_edges(src, dst, CHUNK2, EPT2)
  zrows1 = jnp.zeros((NPAD1 // NTILES, CW), jnp.float32)

  x2d = x.reshape(N, 2, CW).transpose(1, 0, 2).reshape(2 * N, CW)
  agg1 = _segsum2(x2d, e4a, zrows1)
  h_r = _tc1(x, agg1, W1.astype(jnp.bfloat16),
             b1.reshape(NHID // CW, 1, CW))
  agg2 = _segsum8(h_r.reshape(8 * N, CW), e4b, zrows1)
  out = _tc2(h_r, agg2, W2.astype(jnp.bfloat16), b2.reshape(1, NCLASS))
  return out


# speedup vs baseline: 4.4271x; 2.1533x over previous
import functools

import jax
import jax.numpy as jnp
from jax import lax
from jax.experimental import pallas as pl
from jax.experimental.pallas import tpu as pltpu
from jax.experimental.pallas import tpu_sc as plsc

N = 10000
E = 160000
NFEAT = 256
NHID = 1024
NCLASS = 256

NTILES = 16
NCORES = 2
CW = 128


def _make_segsum(nchunks, dtype, chunk, ept_ch, nbuf, npad):
  cp = nchunks // NCORES
  rpt = npad // NTILES
  mesh = plsc.VectorSubcoreMesh(core_axis_name="c", subcore_axis_name="s")

  @functools.partial(
      pl.kernel,
      mesh=mesh,
      out_type=jax.ShapeDtypeStruct((nchunks, npad, CW), dtype),
      scratch_types=[
          pltpu.VMEM((nbuf, 2, chunk), jnp.int32),
          pltpu.VMEM((nbuf, chunk), jnp.int32),
          pltpu.VMEM((nbuf, chunk), jnp.int32),
          [pltpu.VMEM((chunk, CW), dtype) for _ in range(nbuf)],
          pltpu.VMEM_SHARED((npad, CW), dtype),
          [pltpu.SemaphoreType.DMA for _ in range(nbuf)],
          [pltpu.SemaphoreType.DMA for _ in range(nbuf)],
          [pltpu.SemaphoreType.DMA for _ in range(nbuf)],
      ],
  )
  def segsum(table, e4, zrows, out_r, ebuf, gidx_v, dbuf, gbufs, agg_sh,
             se, sg, ss):
    c = lax.axis_index("c")
    s = lax.axis_index("s")

    def start_idx(ch, b):
      pltpu.async_copy(e4.at[s, ch], ebuf.at[b], se[b])

    def wait_idx(ch, b):
      pltpu.make_async_copy(e4.at[s, ch], ebuf.at[b], se[b]).wait()

    def start_gather(b):
      pltpu.async_copy(table.at[gidx_v.at[b]], gbufs[b], sg[b])

    def wait_gather(b):
      pltpu.make_async_copy(table.at[gidx_v.at[b]], gbufs[b], sg[b]).wait()

    def start_scatter(b):
      pltpu.async_copy(gbufs[b], agg_sh.at[dbuf.at[b]], ss[b], add=True)

    def wait_scatter(b):
      pltpu.make_async_copy(gbufs[b], agg_sh.at[dbuf.at[b]], ss[b]).wait()

    def unpack_idx(b, base):
      for k in range(chunk // 16):
        sl = pl.ds(k * 16, 16)
        gidx_v[b, sl] = ebuf[b, 0, sl] + base
        dbuf[b, sl] = ebuf[b, 1, sl]

    for jj in range(cp):
      j = c * cp + jj
      pltpu.sync_copy(zrows, agg_sh.at[pl.ds(s * rpt, rpt)])
      base = j * N
      plsc.subcore_barrier()

      for ch in range(nbuf - 1):
        start_idx(ch, ch)
      wait_idx(0, 0)
      unpack_idx(0, base)
      start_gather(0)

      @pl.loop(0, ept_ch, step=nbuf)
      def chunk_body(chb):
        for bb in range(nbuf):
          ch = chb + bb
          b = bb
          b1 = (bb + 1) % nbuf
          b2 = (bb + nbuf - 1) % nbuf

          @pl.when(ch + 1 < ept_ch)
          def _():
            wait_idx(ch + 1, b1)

            @pl.when(ch >= nbuf - 1)
            def _():
              wait_scatter(b1)

            unpack_idx(b1, base)
            start_gather(b1)

          @pl.when(ch + nbuf - 1 < ept_ch)
          def _():
            start_idx(ch + nbuf - 1, b2)

          wait_gather(b)
          start_scatter(b)

      for ch in range(ept_ch - nbuf, ept_ch):
        wait_scatter(ch % nbuf)
      plsc.subcore_barrier()
      pltpu.sync_copy(
          agg_sh.at[pl.ds(s * rpt, rpt)],
          out_r.at[j, pl.ds(s * rpt, rpt)])

  return segsum


CHUNK1, EPT1, NPAD1 = 112, 90, 10112
_segsum2 = _make_segsum(2, jnp.float32, CHUNK1, EPT1, 3, NPAD1)


def _pack_edges(src, dst, chunk, ept_ch):
  epad = NTILES * ept_ch * chunk
  src_p = jnp.concatenate([src, jnp.zeros((epad - E,), jnp.int32)])
  dst_p = jnp.concatenate([dst, jnp.full((epad - E,), N, jnp.int32)])
  return jnp.stack([src_p.reshape(NTILES, ept_ch, chunk),
                    dst_p.reshape(NTILES, ept_ch, chunk)], axis=2)


def _tc1_body(x_ref, agg_ref, w_ref, b_ref, out_ref):
  a = jnp.concatenate([agg_ref[0], agg_ref[1]], axis=-1)
  xa = (x_ref[...] + a).astype(jnp.bfloat16)
  acc = jnp.dot(xa, w_ref[...], preferred_element_type=jnp.float32)
  out_ref[...] = jnp.maximum(acc + b_ref[0], 0.0)


def _tc1(x, agg1, w1, b1r):
  bn = 10000
  grid = (N // bn, NHID // CW)
  return pl.pallas_call(
      _tc1_body,
      grid=grid,
      in_specs=[
          pl.BlockSpec((bn, NFEAT), lambda i, j: (i, 0)),
          pl.BlockSpec((2, bn, CW), lambda i, j: (0, i, 0)),
          pl.BlockSpec((NFEAT, CW), lambda i, j: (0, j)),
          pl.BlockSpec((1, 1, CW), lambda i, j: (j, 0, 0)),
      ],
      out_specs=pl.BlockSpec((bn, CW), lambda i, j: (i, j)),
      out_shape=jax.ShapeDtypeStruct((N, NHID), jnp.float32),
  )(x, agg1, w1, b1r)


def _tc2_body(h_ref, w_ref, out_ref):
  hb = h_ref[...].astype(jnp.bfloat16)
  out_ref[0] = jnp.dot(hb, w_ref[...], preferred_element_type=jnp.float32)


def _tc2(h, w2):
  bn = 2000
  grid = (N // bn, NCLASS // CW)
  return pl.pallas_call(
      _tc2_body,
      grid=grid,
      in_specs=[
          pl.BlockSpec((bn, NHID), lambda i, j: (i, 0)),
          pl.BlockSpec((NHID, CW), lambda i, j: (0, j)),
      ],
      out_specs=pl.BlockSpec((1, bn, CW), lambda i, j: (j, i, 0)),
      out_shape=jax.ShapeDtypeStruct((NCLASS // CW, N, CW), jnp.float32),
  )(h, w2)


def _tc3_body(y_ref, agg_ref, b_ref, out_ref):
  y = jnp.concatenate([y_ref[0] + agg_ref[0], y_ref[1] + agg_ref[1]],
                      axis=-1)
  out_ref[...] = y + b_ref[...]


def _tc3(y_r, agg2, b2r):
  bn = 2000
  grid = (N // bn,)
  return pl.pallas_call(
      _tc3_body,
      grid=grid,
      in_specs=[
          pl.BlockSpec((2, bn, CW), lambda i: (0, i, 0)),
          pl.BlockSpec((2, bn, CW), lambda i: (0, i, 0)),
          pl.BlockSpec((1, NCLASS), lambda i: (0, 0)),
      ],
      out_specs=pl.BlockSpec((bn, NCLASS), lambda i: (i, 0)),
      out_shape=jax.ShapeDtypeStruct((N, NCLASS), jnp.float32),
  )(y_r, agg2, b2r)


def kernel(x, edge_index, W1, b1, W2, b2):
  src = edge_index[0].astype(jnp.int32)
  dst = edge_index[1].astype(jnp.int32)
  e4a = _pack_edges(src, dst, CHUNK1, EPT1)
  zrows1 = jnp.zeros((NPAD1 // NTILES, CW), jnp.float32)

  x2d = x.reshape(N, 2, CW).transpose(1, 0, 2).reshape(2 * N, CW)
  agg1 = _segsum2(x2d, e4a, zrows1)
  h = _tc1(x, agg1, W1.astype(jnp.bfloat16),
           b1.reshape(NHID // CW, 1, CW))
  y_r = _tc2(h, W2.astype(jnp.bfloat16))
  agg2 = _segsum2(y_r.reshape(2 * N, CW), e4a, zrows1)
  out = _tc3(y_r, agg2, b2.reshape(1, NCLASS))
  return out


# speedup vs baseline: 4.6778x; 1.0566x over previous
import functools

import jax
import jax.numpy as jnp
from jax import lax
from jax.experimental import pallas as pl
from jax.experimental.pallas import tpu as pltpu
from jax.experimental.pallas import tpu_sc as plsc

N = 10000
E = 160000
NFEAT = 256
NHID = 1024
NCLASS = 256

NTILES = 16
NCORES = 2
CW = 128


def _make_segsum(nchunks, dtype, chunk, ept_ch, nbuf, npad):
  cp = nchunks // NCORES
  rpt = npad // NTILES
  mesh = plsc.VectorSubcoreMesh(core_axis_name="c", subcore_axis_name="s")

  @functools.partial(
      pl.kernel,
      mesh=mesh,
      out_type=jax.ShapeDtypeStruct((nchunks, npad, CW), dtype),
      scratch_types=[
          pltpu.VMEM((nbuf, 2, chunk), jnp.int32),
          pltpu.VMEM((nbuf, chunk), jnp.int32),
          pltpu.VMEM((nbuf, chunk), jnp.int32),
          [pltpu.VMEM((chunk, CW), dtype) for _ in range(nbuf)],
          pltpu.VMEM_SHARED((npad, CW), dtype),
          [pltpu.SemaphoreType.DMA for _ in range(nbuf)],
          [pltpu.SemaphoreType.DMA for _ in range(nbuf)],
          [pltpu.SemaphoreType.DMA for _ in range(nbuf)],
      ],
  )
  def segsum(table, e4, zrows, out_r, ebuf, gidx_v, dbuf, gbufs, agg_sh,
             se, sg, ss):
    c = lax.axis_index("c")
    s = lax.axis_index("s")

    def start_idx(ch, b):
      pltpu.async_copy(e4.at[s, ch], ebuf.at[b], se[b])

    def wait_idx(ch, b):
      pltpu.make_async_copy(e4.at[s, ch], ebuf.at[b], se[b]).wait()

    def start_gather(b):
      pltpu.async_copy(table.at[gidx_v.at[b]], gbufs[b], sg[b])

    def wait_gather(b):
      pltpu.make_async_copy(table.at[gidx_v.at[b]], gbufs[b], sg[b]).wait()

    def start_scatter(b):
      pltpu.async_copy(gbufs[b], agg_sh.at[dbuf.at[b]], ss[b], add=True)

    def wait_scatter(b):
      pltpu.make_async_copy(gbufs[b], agg_sh.at[dbuf.at[b]], ss[b]).wait()

    def unpack_idx(b, base):
      for k in range(chunk // 16):
        sl = pl.ds(k * 16, 16)
        gidx_v[b, sl] = ebuf[b, 0, sl] + base
        dbuf[b, sl] = ebuf[b, 1, sl]

    for jj in range(cp):
      j = c * cp + jj
      pltpu.sync_copy(zrows, agg_sh.at[pl.ds(s * rpt, rpt)])
      base = j * N
      plsc.subcore_barrier()

      for ch in range(nbuf - 1):
        start_idx(ch, ch)
      wait_idx(0, 0)
      unpack_idx(0, base)
      start_gather(0)

      @pl.loop(0, ept_ch, step=nbuf)
      def chunk_body(chb):
        for bb in range(nbuf):
          ch = chb + bb
          b = bb
          b1 = (bb + 1) % nbuf
          b2 = (bb + nbuf - 1) % nbuf

          @pl.when(ch + 1 < ept_ch)
          def _():
            wait_idx(ch + 1, b1)

            @pl.when(ch >= nbuf - 1)
            def _():
              wait_scatter(b1)

            unpack_idx(b1, base)
            start_gather(b1)

          @pl.when(ch + nbuf - 1 < ept_ch)
          def _():
            start_idx(ch + nbuf - 1, b2)

          wait_gather(b)
          start_scatter(b)

      for ch in range(ept_ch - nbuf, ept_ch):
        wait_scatter(ch % nbuf)
      plsc.subcore_barrier()
      pltpu.sync_copy(
          agg_sh.at[pl.ds(s * rpt, rpt)],
          out_r.at[j, pl.ds(s * rpt, rpt)])

  return segsum


CHUNK1, EPT1, NPAD1 = 112, 90, 10112
_segsum2 = _make_segsum(2, jnp.float32, CHUNK1, EPT1, 3, NPAD1)


def _pack_edges(src, dst, chunk, ept_ch):
  epad = NTILES * ept_ch * chunk
  src_p = jnp.concatenate([src, jnp.zeros((epad - E,), jnp.int32)])
  dst_p = jnp.concatenate([dst, jnp.full((epad - E,), N, jnp.int32)])
  return jnp.stack([src_p.reshape(NTILES, ept_ch, chunk),
                    dst_p.reshape(NTILES, ept_ch, chunk)], axis=2)


def _tc12_body(x_ref, agg_ref, w1_ref, b1_ref, w2_ref, out_ref):
  a = jnp.concatenate([agg_ref[0], agg_ref[1]], axis=-1)
  xa = (x_ref[...] + a).astype(jnp.bfloat16)
  h = jnp.maximum(
      jnp.dot(xa, w1_ref[...], preferred_element_type=jnp.float32)
      + b1_ref[...], 0.0)
  y = jnp.dot(h.astype(jnp.bfloat16), w2_ref[...],
              preferred_element_type=jnp.float32)
  out_ref[0] = y[:, :CW]
  out_ref[1] = y[:, CW:]


def _tc12(x, agg1, w1, b1r, w2):
  bn = 2000
  grid = (N // bn,)
  return pl.pallas_call(
      _tc12_body,
      grid=grid,
      in_specs=[
          pl.BlockSpec((bn, NFEAT), lambda i: (i, 0)),
          pl.BlockSpec((2, bn, CW), lambda i: (0, i, 0)),
          pl.BlockSpec((NFEAT, NHID), lambda i: (0, 0)),
          pl.BlockSpec((1, NHID), lambda i: (0, 0)),
          pl.BlockSpec((NHID, NCLASS), lambda i: (0, 0)),
      ],
      out_specs=pl.BlockSpec((2, bn, CW), lambda i: (0, i, 0)),
      out_shape=jax.ShapeDtypeStruct((NCLASS // CW, N, CW), jnp.float32),
  )(x, agg1, w1, b1r, w2)


def _tc3_body(y_ref, agg_ref, b_ref, out_ref):
  y = jnp.concatenate([y_ref[0] + agg_ref[0], y_ref[1] + agg_ref[1]],
                      axis=-1)
  out_ref[...] = y + b_ref[...]


def _tc3(y_r, agg2, b2r):
  bn = 2000
  grid = (N // bn,)
  return pl.pallas_call(
      _tc3_body,
      grid=grid,
      in_specs=[
          pl.BlockSpec((2, bn, CW), lambda i: (0, i, 0)),
          pl.BlockSpec((2, bn, CW), lambda i: (0, i, 0)),
          pl.BlockSpec((1, NCLASS), lambda i: (0, 0)),
      ],
      out_specs=pl.BlockSpec((bn, NCLASS), lambda i: (i, 0)),
      out_shape=jax.ShapeDtypeStruct((N, NCLASS), jnp.float32),
  )(y_r, agg2, b2r)


def kernel(x, edge_index, W1, b1, W2, b2):
  src = edge_index[0].astype(jnp.int32)
  dst = edge_index[1].astype(jnp.int32)
  e4a = _pack_edges(src, dst, CHUNK1, EPT1)
  zrows1 = jnp.zeros((NPAD1 // NTILES, CW), jnp.float32)

  x2d = x.reshape(N, 2, CW).transpose(1, 0, 2).reshape(2 * N, CW)
  agg1 = _segsum2(x2d, e4a, zrows1)
  y_r = _tc12(x, agg1, W1.astype(jnp.bfloat16), b1.reshape(1, NHID),
              W2.astype(jnp.bfloat16))
  agg2 = _segsum2(y_r.reshape(2 * N, CW), e4a, zrows1)
  out = _tc3(y_r, agg2, b2.reshape(1, NCLASS))
  return out
